# baseline (speedup 1.0000x reference)
import jax
import jax.numpy as jnp
from jax import lax
from jax.experimental import pallas as pl
from jax.experimental.pallas import tpu as pltpu
from jax.experimental.pallas import tpu_sc as plsc

BATCH = 16384
EMBED_DIM = 64
LANES = 16
N_ROWS = 1000000
ROW_BLOCKS = N_ROWS // LANES

_info = plsc.get_sparse_core_info()
NUM_CORES = _info.num_cores
NUM_SUBCORES = _info.num_subcores
NW = NUM_CORES * NUM_SUBCORES
BPW = BATCH // NW
RCH = 32
NR = BPW // RCH
GPR = RCH // LANES
NIDX = RCH * EMBED_DIM
NGATH = NIDX // 128
GROUPS = BPW // LANES


def _sc_kernel(uid_hbm, iid_hbm, ut4_hbm, it4_hbm, out_hbm,
               uid, iid, uidx, iidx, urows, irows, outv, sem):
    wid = lax.axis_index("s") * NUM_CORES + lax.axis_index("c")
    base = wid * BPW

    pltpu.sync_copy(uid_hbm.at[wid], uid)
    pltpu.sync_copy(iid_hbm.at[wid], iid)

    lane = lax.iota(jnp.int32, LANES)

    def round_body(r, carry):
        for g in range(GPR):
            ubase = uid[r * GPR + g] >> 4
            ibase = iid[r * GPR + g] >> 4
            for d in range(EMBED_DIM):
                off = d * RCH + g * LANES
                uidx[pl.ds(off, LANES)] = ubase + d * ROW_BLOCKS
                iidx[pl.ds(off, LANES)] = ibase + d * ROW_BLOCKS
        for j in range(NGATH):
            src = pl.ds(j * 128, 128)
            pltpu.async_copy(ut4_hbm.at[uidx.at[src]], urows.at[src], sem)
            pltpu.async_copy(it4_hbm.at[iidx.at[src]], irows.at[src], sem)
        for j in range(NGATH):
            src = pl.ds(j * 128, 128)
            pltpu.make_async_copy(ut4_hbm.at[uidx.at[src]],
                                  urows.at[src], sem).wait()
            pltpu.make_async_copy(it4_hbm.at[iidx.at[src]],
                                  irows.at[src], sem).wait()

        for g in range(GPR):
            usub = uid[r * GPR + g] & 15
            isub = iid[r * GPR + g] & 15
            l = g * LANES + lane
            acc0 = jnp.zeros((LANES,), jnp.float32)
            acc1 = jnp.zeros((LANES,), jnp.float32)
            for d in range(0, EMBED_DIM, 2):
                q0 = d * RCH + l
                q1 = (d + 1) * RCH + l
                acc0 = acc0 + (plsc.load_gather(urows, [q0, usub]) *
                               plsc.load_gather(irows, [q0, isub]))
                acc1 = acc1 + (plsc.load_gather(urows, [q1, usub]) *
                               plsc.load_gather(irows, [q1, isub]))
            outv[pl.ds(r * RCH + g * LANES, LANES)] = acc0 + acc1
        return carry

    lax.fori_loop(0, NR, round_body, 0)

    pltpu.sync_copy(outv, out_hbm.at[pl.ds(base, BPW)])


@jax.jit
def kernel(user_ids, item_ids, user_table, item_table):
    uid = user_ids.astype(jnp.int32).reshape(NW, GROUPS, LANES)
    iid = item_ids.astype(jnp.int32).reshape(NW, GROUPS, LANES)
    ut_t, it_t = jax.lax.optimization_barrier((user_table.T, item_table.T))
    ut4 = ut_t.reshape(EMBED_DIM * ROW_BLOCKS, LANES)
    it4 = it_t.reshape(EMBED_DIM * ROW_BLOCKS, LANES)

    mesh = plsc.VectorSubcoreMesh(core_axis_name="c", subcore_axis_name="s")
    run = pl.kernel(
        _sc_kernel,
        out_type=jax.ShapeDtypeStruct((BATCH,), jnp.float32),
        mesh=mesh,
        scratch_types=[
            pltpu.VMEM((GROUPS, LANES), jnp.int32),
            pltpu.VMEM((GROUPS, LANES), jnp.int32),
            pltpu.VMEM((NIDX,), jnp.int32),
            pltpu.VMEM((NIDX,), jnp.int32),
            pltpu.VMEM((NIDX, LANES), jnp.float32),
            pltpu.VMEM((NIDX, LANES), jnp.float32),
            pltpu.VMEM((BPW,), jnp.float32),
            pltpu.SemaphoreType.DMA,
        ],
        compiler_params=pltpu.CompilerParams(
            needs_layout_passes=False, use_tc_tiling_on_sc=False),
    )
    return run(uid, iid, ut4, it4)


# speedup vs baseline: 8.9496x; 8.9496x over previous
import jax
import jax.numpy as jnp
from jax import lax
from jax.experimental import pallas as pl
from jax.experimental.pallas import tpu as pltpu
from jax.experimental.pallas import tpu_sc as plsc

BATCH = 16384
EMBED_DIM = 64
LANES = 16

_info = plsc.get_sparse_core_info()
NUM_CORES = _info.num_cores
NUM_SUBCORES = _info.num_subcores
NW = NUM_CORES * NUM_SUBCORES
BPW = BATCH // NW
CHUNK = 128
NCHUNK = BPW // CHUNK
GROUPS = BPW // LANES


def _sc_kernel(uid_hbm, iid_hbm, ut_hbm, it_hbm, out_hbm,
               uidx, iidx, urows, irows, outv, sem):
    wid = lax.axis_index("s") * NUM_CORES + lax.axis_index("c")
    base = wid * BPW

    pltpu.sync_copy(uid_hbm.at[wid], uidx)
    pltpu.sync_copy(iid_hbm.at[wid], iidx)

    for j in range(NCHUNK):
        dst = pl.ds(j * CHUNK, CHUNK)
        pltpu.async_copy(ut_hbm.at[uidx.at[j]], urows.at[dst], sem)
        pltpu.async_copy(it_hbm.at[iidx.at[j]], irows.at[dst], sem)
    for j in range(NCHUNK):
        dst = pl.ds(j * CHUNK, CHUNK)
        pltpu.make_async_copy(ut_hbm.at[uidx.at[j]], urows.at[dst], sem).wait()
        pltpu.make_async_copy(it_hbm.at[iidx.at[j]], irows.at[dst], sem).wait()

    lane = lax.iota(jnp.int32, LANES)

    def group_body(g, carry):
        rows = g * LANES + lane
        acc0 = jnp.zeros((LANES,), jnp.float32)
        acc1 = jnp.zeros((LANES,), jnp.float32)
        for d in range(0, EMBED_DIM, 2):
            c0 = jnp.full((LANES,), d, jnp.int32)
            c1 = jnp.full((LANES,), d + 1, jnp.int32)
            acc0 = acc0 + (plsc.load_gather(urows, [rows, c0]) *
                           plsc.load_gather(irows, [rows, c0]))
            acc1 = acc1 + (plsc.load_gather(urows, [rows, c1]) *
                           plsc.load_gather(irows, [rows, c1]))
        outv[pl.ds(g * LANES, LANES)] = acc0 + acc1
        return carry

    lax.fori_loop(0, GROUPS, group_body, 0)

    pltpu.sync_copy(outv, out_hbm.at[pl.ds(base, BPW)])


@jax.jit
def kernel(user_ids, item_ids, user_table, item_table):
    uid = user_ids.astype(jnp.int32).reshape(NW, NCHUNK, CHUNK)
    iid = item_ids.astype(jnp.int32).reshape(NW, NCHUNK, CHUNK)

    mesh = plsc.VectorSubcoreMesh(core_axis_name="c", subcore_axis_name="s")
    run = pl.kernel(
        _sc_kernel,
        out_type=jax.ShapeDtypeStruct((BATCH,), jnp.float32),
        mesh=mesh,
        scratch_types=[
            pltpu.VMEM((NCHUNK, CHUNK), jnp.int32),
            pltpu.VMEM((NCHUNK, CHUNK), jnp.int32),
            pltpu.VMEM((BPW, EMBED_DIM), jnp.float32),
            pltpu.VMEM((BPW, EMBED_DIM), jnp.float32),
            pltpu.VMEM((BPW,), jnp.float32),
            pltpu.SemaphoreType.DMA,
        ],
        compiler_params=pltpu.CompilerParams(
            needs_layout_passes=False, use_tc_tiling_on_sc=False),
    )
    return run(uid, iid, user_table, item_table)
